# Initial kernel scaffold; baseline (speedup 1.0000x reference)
import jax
import jax.numpy as jnp
from jax.experimental import pallas as pl


def kernel(x, edge_index, W1, att_src1, att_dst1, b1, W2, att_src2, att_dst2, b2):
    raise NotImplementedError("write your pallas kernel here")


# speedup vs baseline: 43.4684x; 43.4684x over previous
import functools

import jax
import jax.numpy as jnp
from jax import lax
from jax.experimental import pallas as pl
from jax.experimental.pallas import tpu as pltpu
from jax.experimental.pallas import tpu_sc as plsc

_NC = 2
_NS = 16
_NW = _NC * _NS
_K = 128


def _mm_att(x, W, a_src, a_dst, blk):
    n, d_in = x.shape
    d = W.shape[1]

    def body(x_ref, w_ref, s_ref, t_ref, h_ref, as_ref, ad_ref):
        h = jnp.dot(x_ref[...], w_ref[...], preferred_element_type=jnp.float32)
        h_ref[...] = h
        as_ref[...] = jnp.dot(h, s_ref[...], preferred_element_type=jnp.float32)
        ad_ref[...] = jnp.dot(h, t_ref[...], preferred_element_type=jnp.float32)

    return pl.pallas_call(
        body,
        grid=(n // blk,),
        in_specs=[
            pl.BlockSpec((blk, d_in), lambda i: (i, 0)),
            pl.BlockSpec((d_in, d), lambda i: (0, 0)),
            pl.BlockSpec((d, 1), lambda i: (0, 0)),
            pl.BlockSpec((d, 1), lambda i: (0, 0)),
        ],
        out_specs=[
            pl.BlockSpec((blk, d), lambda i: (i, 0)),
            pl.BlockSpec((blk, 1), lambda i: (i, 0)),
            pl.BlockSpec((blk, 1), lambda i: (i, 0)),
        ],
        out_shape=[
            jax.ShapeDtypeStruct((n, d), jnp.float32),
            jax.ShapeDtypeStruct((n, 1), jnp.float32),
            jax.ShapeDtypeStruct((n, 1), jnp.float32),
        ],
    )(x, W, a_src, a_dst)


def _norm_relu_mm_att(p, dn, b, W, a_src, a_dst, blk):
    _, n, d = p.shape
    d2 = W.shape[1]

    def body(p_ref, dn_ref, b_ref, w_ref, s_ref, t_ref, h_ref, as_ref, ad_ref):
        acc = p_ref[0] + p_ref[1]
        den = dn_ref[0] + dn_ref[1] + 1e-16
        z = jnp.maximum(acc / den + b_ref[...], 0.0)
        h = jnp.dot(z, w_ref[...], preferred_element_type=jnp.float32)
        h_ref[...] = h
        as_ref[...] = jnp.dot(h, s_ref[...], preferred_element_type=jnp.float32)
        ad_ref[...] = jnp.dot(h, t_ref[...], preferred_element_type=jnp.float32)

    return pl.pallas_call(
        body,
        grid=(n // blk,),
        in_specs=[
            pl.BlockSpec((2, blk, d), lambda i: (0, i, 0)),
            pl.BlockSpec((2, blk, 1), lambda i: (0, i, 0)),
            pl.BlockSpec((1, d), lambda i: (0, 0)),
            pl.BlockSpec((d, d2), lambda i: (0, 0)),
            pl.BlockSpec((d2, 1), lambda i: (0, 0)),
            pl.BlockSpec((d2, 1), lambda i: (0, 0)),
        ],
        out_specs=[
            pl.BlockSpec((blk, d2), lambda i: (i, 0)),
            pl.BlockSpec((blk, 1), lambda i: (i, 0)),
            pl.BlockSpec((blk, 1), lambda i: (i, 0)),
        ],
        out_shape=[
            jax.ShapeDtypeStruct((n, d2), jnp.float32),
            jax.ShapeDtypeStruct((n, 1), jnp.float32),
            jax.ShapeDtypeStruct((n, 1), jnp.float32),
        ],
    )(p, dn, b, W, a_src, a_dst)


def _norm_bias(p, dn, b, blk):
    _, n, d = p.shape

    def body(p_ref, dn_ref, b_ref, o_ref):
        den = dn_ref[0] + dn_ref[1] + 1e-16
        o_ref[...] = (p_ref[0] + p_ref[1]) / den + b_ref[...]

    return pl.pallas_call(
        body,
        grid=(n // blk,),
        in_specs=[
            pl.BlockSpec((2, blk, d), lambda i: (0, i, 0)),
            pl.BlockSpec((2, blk, 1), lambda i: (0, i, 0)),
            pl.BlockSpec((1, d), lambda i: (0, 0)),
        ],
        out_specs=pl.BlockSpec((blk, d), lambda i: (i, 0)),
        out_shape=jax.ShapeDtypeStruct((n, d), jnp.float32),
    )(p, dn, b)


@functools.lru_cache(maxsize=None)
def _make_edge_kernel(n, d, cpw, e_real):
    mesh = plsc.VectorSubcoreMesh(core_axis_name="c", subcore_axis_name="s")
    rps = (n // _NS) // 16 * 16
    rem = n - _NS * rps
    ngrp = _K // 16

    @functools.partial(
        pl.kernel,
        out_type=(
            jax.ShapeDtypeStruct((_NC, n, d), jnp.float32),
            jax.ShapeDtypeStruct((_NC * n,), jnp.float32),
        ),
        mesh=mesh,
        compiler_params=pltpu.CompilerParams(needs_layout_passes=False),
        scratch_types=[
            [pltpu.VMEM((1, _K), jnp.int32) for _ in range(3)],
            [pltpu.VMEM((1, _K), jnp.int32) for _ in range(3)],
            [pltpu.VMEM((_K,), jnp.float32) for _ in range(2)],
            [pltpu.VMEM((_K,), jnp.float32) for _ in range(2)],
            [pltpu.VMEM((_K,), jnp.float32) for _ in range(2)],
            [pltpu.VMEM((_K, d), jnp.float32) for _ in range(2)],
            pltpu.VMEM_SHARED((n, d), jnp.float32),
            pltpu.VMEM_SHARED((n,), jnp.float32),
            pltpu.VMEM((rps,), jnp.float32),
            [pltpu.SemaphoreType.DMA for _ in range(3)],
            [pltpu.SemaphoreType.DMA for _ in range(2)],
            [pltpu.SemaphoreType.DMA for _ in range(2)],
            [pltpu.SemaphoreType.DMA for _ in range(2)],
            [pltpu.SemaphoreType.DMA for _ in range(2)],
        ],
    )
    def ek(h_hbm, as_hbm, ad_hbm, src_hbm, dst_hbm, zrow_hbm,
           out_hbm, den_hbm,
           si3, di3, ag2, dg2, w2, rows2, acc_sh, den_sh, bnc_v,
           isem, asem, dsem, gsem, ssem):
        c = lax.axis_index("c")
        s = lax.axis_index("s")
        wid = c * _NS + s
        base = wid * cpw
        pltpu.sync_copy(zrow_hbm.at[pl.ds(s * rps, rps)],
                        acc_sh.at[pl.ds(s * rps, rps)])

        def zb(j, carry):
            bnc_v[pl.ds(j * 16, 16)] = jnp.zeros((16,), jnp.float32)
            return carry

        lax.fori_loop(0, rps // 16, zb, 0)
        pltpu.sync_copy(bnc_v, den_sh.at[pl.ds(s * rps, rps)])
        if rem:
            @pl.when(s == 0)
            def _():
                pltpu.sync_copy(zrow_hbm.at[pl.ds(_NS * rps, rem)],
                                acc_sh.at[pl.ds(_NS * rps, rem)])
                pltpu.sync_copy(bnc_v.at[pl.ds(0, rem)],
                                den_sh.at[pl.ds(_NS * rps, rem)])
        plsc.subcore_barrier()

        lane = lax.iota(jnp.int32, 16)

        def start_idx(i, t):
            pltpu.async_copy(src_hbm.at[pl.ds(base + i, 1)], si3[t], isem[t])
            pltpu.async_copy(dst_hbm.at[pl.ds(base + i, 1)], di3[t], isem[t])

        def wait_idx(t):
            pltpu.make_async_copy(src_hbm.at[pl.ds(0, 1)], si3[t],
                                  isem[t]).wait()
            pltpu.make_async_copy(dst_hbm.at[pl.ds(0, 1)], di3[t],
                                  isem[t]).wait()

        def start_scores(b, t):
            pltpu.async_copy(as_hbm.at[si3[t].at[0]], ag2[b], asem[b])
            pltpu.async_copy(ad_hbm.at[di3[t].at[0]], dg2[b], dsem[b])

        def wait_scores(b):
            pltpu.make_async_copy(as_hbm.at[si3[0].at[0]], ag2[b],
                                  asem[b]).wait()
            pltpu.make_async_copy(ad_hbm.at[di3[0].at[0]], dg2[b],
                                  dsem[b]).wait()

        def start_rows(b, t):
            pltpu.async_copy(h_hbm.at[si3[t].at[0]], rows2[b], gsem[b])

        def wait_rows(b):
            pltpu.make_async_copy(h_hbm.at[si3[0].at[0]], rows2[b],
                                  gsem[b]).wait()

        def start_scatter(b, t):
            pltpu.async_copy(rows2[b], acc_sh.at[di3[t].at[0]], ssem[b],
                             add=True)

        def wait_scatter(b):
            pltpu.make_async_copy(rows2[b], acc_sh.at[di3[0].at[0]],
                                  ssem[b]).wait()

        def weights(i, b, t):
            def grp(j, carry):
                e = ag2[b][pl.ds(j * 16, 16)] + dg2[b][pl.ds(j * 16, 16)]
                e = jnp.maximum(e, 0.2 * e)
                w = jnp.exp(e)
                eid = (base + i) * _K + j * 16 + lane
                w = jnp.where(eid < e_real, w, 0.0)
                w2[b][pl.ds(j * 16, 16)] = w
                return carry

            lax.fori_loop(0, ngrp, grp, 0)

        def scale(b):
            rows_v, w_v = rows2[b], w2[b]

            def body(eix, carry):
                wvec = plsc.load_gather(
                    w_v, [jnp.full((16,), eix, jnp.int32)])
                for dd in range(d // 16):
                    sl = pl.ds(dd * 16, 16)
                    rows_v[eix, sl] = rows_v[eix, sl] * wvec
                return carry

            lax.fori_loop(0, _K, body, 0)

        start_idx(0, 0)
        wait_idx(0)
        start_scores(0, 0)
        start_rows(0, 0)
        start_idx(1, 1)

        def body_pair(i, carry):
            for u in range(6):
                ci = i + u
                b = u & 1
                bn = (u + 1) & 1
                t = u % 3
                tn = (u + 1) % 3
                tnn = (u + 2) % 3
                wait_scores(b)
                weights(ci, b, t)
                @pl.when(ci + 1 < cpw)
                def _():
                    wait_idx(tn)
                    start_scores(bn, tn)

                @pl.when(ci >= 1)
                def _():
                    wait_scatter(bn)

                @pl.when(ci + 1 < cpw)
                def _():
                    start_rows(bn, tn)

                @pl.when(ci + 2 < cpw)
                def _():
                    start_idx(ci + 2, tnn)
                wait_rows(b)
                scale(b)
                start_scatter(b, t)
                pltpu.sync_copy(w2[b], den_sh.at[di3[t].at[0]], add=True)
            return carry

        lax.fori_loop(0, cpw // 6, lambda k, cr: body_pair(k * 6, cr), 0)
        wait_scatter((cpw - 1) & 1)
        plsc.subcore_barrier()
        pltpu.sync_copy(acc_sh.at[pl.ds(s * rps, rps)],
                        out_hbm.at[c, pl.ds(s * rps, rps)])
        pltpu.sync_copy(den_sh.at[pl.ds(s * rps, rps)], bnc_v)
        pltpu.sync_copy(bnc_v, den_hbm.at[pl.ds(c * n + s * rps, rps)])
        if rem:
            @pl.when(s == 0)
            def _():
                pltpu.sync_copy(acc_sh.at[pl.ds(_NS * rps, rem)],
                                out_hbm.at[c, pl.ds(_NS * rps, rem)])
                pltpu.sync_copy(den_sh.at[pl.ds(_NS * rps, rem)],
                                bnc_v.at[pl.ds(0, rem)])
                pltpu.sync_copy(bnc_v.at[pl.ds(0, rem)],
                                den_hbm.at[pl.ds(c * n + _NS * rps, rem)])

    return ek


def kernel(x, edge_index, W1, att_src1, att_dst1, b1, W2, att_src2, att_dst2, b2):
    n, d_in = x.shape
    d_hid = W1.shape[1]
    d_out = W2.shape[1]
    e_edges = edge_index.shape[1]
    e_real = e_edges + n

    cpw = -(-e_real // (_NW * _K))
    cpw = max(6, -(-cpw // 6) * 6)
    total = _NW * cpw * _K
    pad = total - e_real

    loop = jnp.arange(n, dtype=jnp.int32)
    fill = jnp.arange(pad, dtype=jnp.int32) % n
    src = jnp.concatenate([edge_index[0].astype(jnp.int32), loop, fill])
    dst = jnp.concatenate([edge_index[1].astype(jnp.int32), loop, fill])
    src = src.reshape(total // _K, _K)
    dst = dst.reshape(total // _K, _K)

    zrow = jnp.zeros((n, d_hid), jnp.float32)

    blk = 2000
    ek = _make_edge_kernel(n, d_hid, cpw, e_real)

    h1, as1, ad1 = _mm_att(x, W1, att_src1.reshape(d_hid, 1),
                           att_dst1.reshape(d_hid, 1), blk)
    p1, dn1 = ek(h1, as1.reshape(n), ad1.reshape(n), src, dst, zrow)

    h2, as2, ad2 = _norm_relu_mm_att(p1, dn1.reshape(_NC, n, 1),
                                     b1.reshape(1, d_hid), W2,
                                     att_src2.reshape(d_out, 1),
                                     att_dst2.reshape(d_out, 1), blk)
    p2, dn2 = ek(h2, as2.reshape(n), ad2.reshape(n), src, dst, zrow)

    return _norm_bias(p2, dn2.reshape(_NC, n, 1), b2.reshape(1, d_out), blk)


# speedup vs baseline: 49.5735x; 1.1404x over previous
import functools

import jax
import jax.numpy as jnp
from jax import lax
from jax.experimental import pallas as pl
from jax.experimental.pallas import tpu as pltpu
from jax.experimental.pallas import tpu_sc as plsc

_NC = 2
_NS = 16
_NW = _NC * _NS
_K = 128


def _mm_att(x, W, a_src, a_dst, blk):
    n, d_in = x.shape
    d = W.shape[1]

    def body(x_ref, w_ref, s_ref, t_ref, h_ref, as_ref, ad_ref):
        h = jnp.dot(x_ref[...], w_ref[...], preferred_element_type=jnp.float32)
        h_ref[...] = h
        as_ref[...] = jnp.dot(h, s_ref[...], preferred_element_type=jnp.float32)
        ad_ref[...] = jnp.dot(h, t_ref[...], preferred_element_type=jnp.float32)

    return pl.pallas_call(
        body,
        grid=(n // blk,),
        in_specs=[
            pl.BlockSpec((blk, d_in), lambda i: (i, 0)),
            pl.BlockSpec((d_in, d), lambda i: (0, 0)),
            pl.BlockSpec((d, 1), lambda i: (0, 0)),
            pl.BlockSpec((d, 1), lambda i: (0, 0)),
        ],
        out_specs=[
            pl.BlockSpec((blk, d), lambda i: (i, 0)),
            pl.BlockSpec((blk, 1), lambda i: (i, 0)),
            pl.BlockSpec((blk, 1), lambda i: (i, 0)),
        ],
        out_shape=[
            jax.ShapeDtypeStruct((n, d), jnp.float32),
            jax.ShapeDtypeStruct((n, 1), jnp.float32),
            jax.ShapeDtypeStruct((n, 1), jnp.float32),
        ],
    )(x, W, a_src, a_dst)


def _norm_relu_mm_att(p, dn, b, W, a_src, a_dst, blk):
    _, n, d = p.shape
    d2 = W.shape[1]

    def body(p_ref, dn_ref, b_ref, w_ref, s_ref, t_ref, h_ref, as_ref, ad_ref):
        acc = p_ref[0] + p_ref[1]
        den = dn_ref[0] + dn_ref[1] + 1e-16
        z = jnp.maximum(acc / den + b_ref[...], 0.0)
        h = jnp.dot(z, w_ref[...], preferred_element_type=jnp.float32)
        h_ref[...] = h
        as_ref[...] = jnp.dot(h, s_ref[...], preferred_element_type=jnp.float32)
        ad_ref[...] = jnp.dot(h, t_ref[...], preferred_element_type=jnp.float32)

    return pl.pallas_call(
        body,
        grid=(n // blk,),
        in_specs=[
            pl.BlockSpec((2, blk, d), lambda i: (0, i, 0)),
            pl.BlockSpec((2, blk, 1), lambda i: (0, i, 0)),
            pl.BlockSpec((1, d), lambda i: (0, 0)),
            pl.BlockSpec((d, d2), lambda i: (0, 0)),
            pl.BlockSpec((d2, 1), lambda i: (0, 0)),
            pl.BlockSpec((d2, 1), lambda i: (0, 0)),
        ],
        out_specs=[
            pl.BlockSpec((blk, d2), lambda i: (i, 0)),
            pl.BlockSpec((blk, 1), lambda i: (i, 0)),
            pl.BlockSpec((blk, 1), lambda i: (i, 0)),
        ],
        out_shape=[
            jax.ShapeDtypeStruct((n, d2), jnp.float32),
            jax.ShapeDtypeStruct((n, 1), jnp.float32),
            jax.ShapeDtypeStruct((n, 1), jnp.float32),
        ],
    )(p, dn, b, W, a_src, a_dst)


def _norm_bias(p, dn, b, blk):
    _, n, d = p.shape

    def body(p_ref, dn_ref, b_ref, o_ref):
        den = dn_ref[0] + dn_ref[1] + 1e-16
        o_ref[...] = (p_ref[0] + p_ref[1]) / den + b_ref[...]

    return pl.pallas_call(
        body,
        grid=(n // blk,),
        in_specs=[
            pl.BlockSpec((2, blk, d), lambda i: (0, i, 0)),
            pl.BlockSpec((2, blk, 1), lambda i: (0, i, 0)),
            pl.BlockSpec((1, d), lambda i: (0, 0)),
        ],
        out_specs=pl.BlockSpec((blk, d), lambda i: (i, 0)),
        out_shape=jax.ShapeDtypeStruct((n, d), jnp.float32),
    )(p, dn, b)


@functools.lru_cache(maxsize=None)
def _make_edge_kernel(n, d, cpw, e_real):
    mesh = plsc.VectorSubcoreMesh(core_axis_name="c", subcore_axis_name="s")
    rps = (n // _NS) // 16 * 16
    rem = n - _NS * rps
    ngrp = _K // 16

    @functools.partial(
        pl.kernel,
        out_type=(
            jax.ShapeDtypeStruct((_NC, n, d), jnp.float32),
            jax.ShapeDtypeStruct((_NC * n,), jnp.float32),
        ),
        mesh=mesh,
        compiler_params=pltpu.CompilerParams(needs_layout_passes=False),
        scratch_types=[
            [pltpu.VMEM((1, _K), jnp.int32) for _ in range(3)],
            [pltpu.VMEM((1, _K), jnp.int32) for _ in range(3)],
            [pltpu.VMEM((_K,), jnp.float32) for _ in range(2)],
            [pltpu.VMEM((_K,), jnp.float32) for _ in range(2)],
            [pltpu.VMEM((_K,), jnp.float32) for _ in range(2)],
            [pltpu.VMEM((_K, d), jnp.float32) for _ in range(2)],
            pltpu.VMEM_SHARED((n, d), jnp.float32),
            pltpu.VMEM_SHARED((n,), jnp.float32),
            pltpu.VMEM((rps,), jnp.float32),
            [pltpu.SemaphoreType.DMA for _ in range(3)],
            [pltpu.SemaphoreType.DMA for _ in range(2)],
            [pltpu.SemaphoreType.DMA for _ in range(2)],
            [pltpu.SemaphoreType.DMA for _ in range(2)],
            [pltpu.SemaphoreType.DMA for _ in range(2)],
            [pltpu.SemaphoreType.DMA for _ in range(2)],
        ],
    )
    def ek(h_hbm, as_hbm, ad_hbm, src_hbm, dst_hbm, zrow_hbm,
           out_hbm, den_hbm,
           si3, di3, ag2, dg2, w2, rows2, acc_sh, den_sh, bnc_v,
           isem, asem, dsem, gsem, ssem, wsem):
        c = lax.axis_index("c")
        s = lax.axis_index("s")
        wid = c * _NS + s
        base = wid * cpw
        pltpu.sync_copy(zrow_hbm.at[pl.ds(s * rps, rps)],
                        acc_sh.at[pl.ds(s * rps, rps)])

        def zb(j, carry):
            bnc_v[pl.ds(j * 16, 16)] = jnp.zeros((16,), jnp.float32)
            return carry

        lax.fori_loop(0, rps // 16, zb, 0)
        pltpu.sync_copy(bnc_v, den_sh.at[pl.ds(s * rps, rps)])
        if rem:
            @pl.when(s == 0)
            def _():
                pltpu.sync_copy(zrow_hbm.at[pl.ds(_NS * rps, rem)],
                                acc_sh.at[pl.ds(_NS * rps, rem)])
                pltpu.sync_copy(bnc_v.at[pl.ds(0, rem)],
                                den_sh.at[pl.ds(_NS * rps, rem)])
        plsc.subcore_barrier()

        lane = lax.iota(jnp.int32, 16)

        def start_idx(i, t):
            pltpu.async_copy(src_hbm.at[pl.ds(base + i, 1)], si3[t], isem[t])
            pltpu.async_copy(dst_hbm.at[pl.ds(base + i, 1)], di3[t], isem[t])

        def wait_idx(t):
            pltpu.make_async_copy(src_hbm.at[pl.ds(0, 1)], si3[t],
                                  isem[t]).wait()
            pltpu.make_async_copy(dst_hbm.at[pl.ds(0, 1)], di3[t],
                                  isem[t]).wait()

        def start_scores(b, t):
            pltpu.async_copy(as_hbm.at[si3[t].at[0]], ag2[b], asem[b])
            pltpu.async_copy(ad_hbm.at[di3[t].at[0]], dg2[b], dsem[b])

        def wait_scores(b):
            pltpu.make_async_copy(as_hbm.at[si3[0].at[0]], ag2[b],
                                  asem[b]).wait()
            pltpu.make_async_copy(ad_hbm.at[di3[0].at[0]], dg2[b],
                                  dsem[b]).wait()

        def start_rows(b, t):
            pltpu.async_copy(h_hbm.at[si3[t].at[0]], rows2[b], gsem[b])

        def wait_rows(b):
            pltpu.make_async_copy(h_hbm.at[si3[0].at[0]], rows2[b],
                                  gsem[b]).wait()

        def start_scatter(b, t):
            pltpu.async_copy(rows2[b], acc_sh.at[di3[t].at[0]], ssem[b],
                             add=True)

        def wait_scatter(b):
            pltpu.make_async_copy(rows2[b], acc_sh.at[di3[0].at[0]],
                                  ssem[b]).wait()

        def start_den(b, t):
            pltpu.async_copy(w2[b], den_sh.at[di3[t].at[0]], wsem[b],
                             add=True)

        def wait_den(b):
            pltpu.make_async_copy(w2[b], den_sh.at[di3[0].at[0]],
                                  wsem[b]).wait()

        def weights(i, b, t):
            @plsc.parallel_loop(0, ngrp, unroll=2)
            def grp(j):
                e = ag2[b][pl.ds(j * 16, 16)] + dg2[b][pl.ds(j * 16, 16)]
                e = jnp.maximum(e, 0.2 * e)
                w = jnp.exp(e)
                eid = (base + i) * _K + j * 16 + lane
                w = jnp.where(eid < e_real, w, 0.0)
                w2[b][pl.ds(j * 16, 16)] = w

        def scale(b):
            rows_v, w_v = rows2[b], w2[b]

            @plsc.parallel_loop(0, _K, unroll=4)
            def body(eix):
                wvec = plsc.load_gather(
                    w_v, [jnp.full((16,), eix, jnp.int32)])
                for dd in range(d // 16):
                    sl = pl.ds(dd * 16, 16)
                    rows_v[eix, sl] = rows_v[eix, sl] * wvec

        start_idx(0, 0)
        wait_idx(0)
        start_scores(0, 0)
        start_rows(0, 0)
        start_idx(1, 1)

        def body_pair(i, carry):
            for u in range(6):
                ci = i + u
                b = u & 1
                bn = (u + 1) & 1
                t = u % 3
                tn = (u + 1) % 3
                tnn = (u + 2) % 3
                wait_scores(b)
                weights(ci, b, t)
                @pl.when(ci + 1 < cpw)
                def _():
                    wait_idx(tn)
                    start_scores(bn, tn)

                @pl.when(ci >= 1)
                def _():
                    wait_scatter(bn)
                    wait_den(bn)

                @pl.when(ci + 1 < cpw)
                def _():
                    start_rows(bn, tn)

                @pl.when(ci + 2 < cpw)
                def _():
                    start_idx(ci + 2, tnn)
                wait_rows(b)
                scale(b)
                start_scatter(b, t)
                start_den(b, t)
            return carry

        lax.fori_loop(0, cpw // 6, lambda k, cr: body_pair(k * 6, cr), 0)
        wait_scatter((cpw - 1) & 1)
        wait_den((cpw - 1) & 1)
        plsc.subcore_barrier()
        pltpu.sync_copy(acc_sh.at[pl.ds(s * rps, rps)],
                        out_hbm.at[c, pl.ds(s * rps, rps)])
        pltpu.sync_copy(den_sh.at[pl.ds(s * rps, rps)], bnc_v)
        pltpu.sync_copy(bnc_v, den_hbm.at[pl.ds(c * n + s * rps, rps)])
        if rem:
            @pl.when(s == 0)
            def _():
                pltpu.sync_copy(acc_sh.at[pl.ds(_NS * rps, rem)],
                                out_hbm.at[c, pl.ds(_NS * rps, rem)])
                pltpu.sync_copy(den_sh.at[pl.ds(_NS * rps, rem)],
                                bnc_v.at[pl.ds(0, rem)])
                pltpu.sync_copy(bnc_v.at[pl.ds(0, rem)],
                                den_hbm.at[pl.ds(c * n + _NS * rps, rem)])

    return ek


def kernel(x, edge_index, W1, att_src1, att_dst1, b1, W2, att_src2, att_dst2, b2):
    n, d_in = x.shape
    d_hid = W1.shape[1]
    d_out = W2.shape[1]
    e_edges = edge_index.shape[1]
    e_real = e_edges + n

    cpw = -(-e_real // (_NW * _K))
    cpw = max(6, -(-cpw // 6) * 6)
    total = _NW * cpw * _K
    pad = total - e_real

    loop = jnp.arange(n, dtype=jnp.int32)
    fill = jnp.arange(pad, dtype=jnp.int32) % n
    src = jnp.concatenate([edge_index[0].astype(jnp.int32), loop, fill])
    dst = jnp.concatenate([edge_index[1].astype(jnp.int32), loop, fill])
    src = src.reshape(total // _K, _K)
    dst = dst.reshape(total // _K, _K)

    zrow = jnp.zeros((n, d_hid), jnp.float32)

    blk = 2000
    ek = _make_edge_kernel(n, d_hid, cpw, e_real)

    h1, as1, ad1 = _mm_att(x, W1, att_src1.reshape(d_hid, 1),
                           att_dst1.reshape(d_hid, 1), blk)
    p1, dn1 = ek(h1, as1.reshape(n), ad1.reshape(n), src, dst, zrow)

    h2, as2, ad2 = _norm_relu_mm_att(p1, dn1.reshape(_NC, n, 1),
                                     b1.reshape(1, d_hid), W2,
                                     att_src2.reshape(d_out, 1),
                                     att_dst2.reshape(d_out, 1), blk)
    p2, dn2 = ek(h2, as2.reshape(n), ad2.reshape(n), src, dst, zrow)

    return _norm_bias(p2, dn2.reshape(_NC, n, 1), b2.reshape(1, d_out), blk)


# speedup vs baseline: 50.0210x; 1.0090x over previous
import functools

import jax
import jax.numpy as jnp
from jax import lax
from jax.experimental import pallas as pl
from jax.experimental.pallas import tpu as pltpu
from jax.experimental.pallas import tpu_sc as plsc

_NC = 2
_NS = 16
_NW = _NC * _NS
_K = 128


def _mm_att(x, W, a_src, a_dst, blk):
    n, d_in = x.shape
    d = W.shape[1]

    def body(x_ref, w_ref, s_ref, t_ref, h_ref, as_ref, ad_ref):
        h = jnp.dot(x_ref[...], w_ref[...], preferred_element_type=jnp.float32)
        h_ref[...] = h
        as_ref[...] = jnp.dot(h, s_ref[...], preferred_element_type=jnp.float32)
        ad_ref[...] = jnp.dot(h, t_ref[...], preferred_element_type=jnp.float32)

    return pl.pallas_call(
        body,
        grid=(n // blk,),
        in_specs=[
            pl.BlockSpec((blk, d_in), lambda i: (i, 0)),
            pl.BlockSpec((d_in, d), lambda i: (0, 0)),
            pl.BlockSpec((d, 1), lambda i: (0, 0)),
            pl.BlockSpec((d, 1), lambda i: (0, 0)),
        ],
        out_specs=[
            pl.BlockSpec((blk, d), lambda i: (i, 0)),
            pl.BlockSpec((blk, 1), lambda i: (i, 0)),
            pl.BlockSpec((blk, 1), lambda i: (i, 0)),
        ],
        out_shape=[
            jax.ShapeDtypeStruct((n, d), jnp.float32),
            jax.ShapeDtypeStruct((n, 1), jnp.float32),
            jax.ShapeDtypeStruct((n, 1), jnp.float32),
        ],
    )(x, W, a_src, a_dst)


def _norm_relu_mm_att(p, dn, b, W, a_src, a_dst, blk):
    _, n, d = p.shape
    d2 = W.shape[1]

    def body(p_ref, dn_ref, b_ref, w_ref, s_ref, t_ref, h_ref, as_ref, ad_ref):
        acc = p_ref[0] + p_ref[1]
        den = dn_ref[0] + dn_ref[1] + 1e-16
        z = jnp.maximum(acc / den + b_ref[...], 0.0)
        h = jnp.dot(z, w_ref[...], preferred_element_type=jnp.float32)
        h_ref[...] = h
        as_ref[...] = jnp.dot(h, s_ref[...], preferred_element_type=jnp.float32)
        ad_ref[...] = jnp.dot(h, t_ref[...], preferred_element_type=jnp.float32)

    return pl.pallas_call(
        body,
        grid=(n // blk,),
        in_specs=[
            pl.BlockSpec((2, blk, d), lambda i: (0, i, 0)),
            pl.BlockSpec((2, blk, 1), lambda i: (0, i, 0)),
            pl.BlockSpec((1, d), lambda i: (0, 0)),
            pl.BlockSpec((d, d2), lambda i: (0, 0)),
            pl.BlockSpec((d2, 1), lambda i: (0, 0)),
            pl.BlockSpec((d2, 1), lambda i: (0, 0)),
        ],
        out_specs=[
            pl.BlockSpec((blk, d2), lambda i: (i, 0)),
            pl.BlockSpec((blk, 1), lambda i: (i, 0)),
            pl.BlockSpec((blk, 1), lambda i: (i, 0)),
        ],
        out_shape=[
            jax.ShapeDtypeStruct((n, d2), jnp.float32),
            jax.ShapeDtypeStruct((n, 1), jnp.float32),
            jax.ShapeDtypeStruct((n, 1), jnp.float32),
        ],
    )(p, dn, b, W, a_src, a_dst)


def _norm_bias(p, dn, b, blk):
    _, n, d = p.shape

    def body(p_ref, dn_ref, b_ref, o_ref):
        den = dn_ref[0] + dn_ref[1] + 1e-16
        o_ref[...] = (p_ref[0] + p_ref[1]) / den + b_ref[...]

    return pl.pallas_call(
        body,
        grid=(n // blk,),
        in_specs=[
            pl.BlockSpec((2, blk, d), lambda i: (0, i, 0)),
            pl.BlockSpec((2, blk, 1), lambda i: (0, i, 0)),
            pl.BlockSpec((1, d), lambda i: (0, 0)),
        ],
        out_specs=pl.BlockSpec((blk, d), lambda i: (i, 0)),
        out_shape=jax.ShapeDtypeStruct((n, d), jnp.float32),
    )(p, dn, b)


@functools.lru_cache(maxsize=None)
def _make_edge_kernel(n, d, cpw, e_real):
    mesh = plsc.VectorSubcoreMesh(core_axis_name="c", subcore_axis_name="s")
    rps = (n // _NS) // 16 * 16
    rem = n - _NS * rps
    ngrp = _K // 16

    @functools.partial(
        pl.kernel,
        out_type=(
            jax.ShapeDtypeStruct((_NC, n, d), jnp.float32),
            jax.ShapeDtypeStruct((_NC * n,), jnp.float32),
        ),
        mesh=mesh,
        compiler_params=pltpu.CompilerParams(needs_layout_passes=False),
        scratch_types=[
            [pltpu.VMEM((1, _K), jnp.int32) for _ in range(3)],
            [pltpu.VMEM((1, _K), jnp.int32) for _ in range(3)],
            [pltpu.VMEM((_K,), jnp.float32) for _ in range(2)],
            [pltpu.VMEM((_K,), jnp.float32) for _ in range(2)],
            [pltpu.VMEM((_K,), jnp.float32) for _ in range(2)],
            [pltpu.VMEM((_K, d), jnp.float32) for _ in range(2)],
            pltpu.VMEM_SHARED((n, d), jnp.float32),
            pltpu.VMEM_SHARED((n,), jnp.float32),
            pltpu.VMEM((rps,), jnp.float32),
            [pltpu.SemaphoreType.DMA for _ in range(3)],
            [pltpu.SemaphoreType.DMA for _ in range(2)],
            [pltpu.SemaphoreType.DMA for _ in range(2)],
            [pltpu.SemaphoreType.DMA for _ in range(2)],
            [pltpu.SemaphoreType.DMA for _ in range(2)],
            [pltpu.SemaphoreType.DMA for _ in range(2)],
        ],
    )
    def ek(h_hbm, as_hbm, ad_hbm, src_hbm, dst_hbm,
           out_hbm, den_hbm,
           si3, di3, ag2, dg2, w2, rows2, acc_sh, den_sh, bnc_v,
           isem, asem, dsem, gsem, ssem, wsem):
        c = lax.axis_index("c")
        s = lax.axis_index("s")
        wid = c * _NS + s
        base = wid * cpw
        zrows = rows2[0]

        @plsc.parallel_loop(0, _K, unroll=4)
        def _(e):
            for dd in range(d // 16):
                zrows[e, pl.ds(dd * 16, 16)] = jnp.zeros((16,), jnp.float32)

        def zb(j, carry):
            bnc_v[pl.ds(j * 16, 16)] = jnp.zeros((16,), jnp.float32)
            return carry

        lax.fori_loop(0, rps // 16, zb, 0)
        for k in range(rps // _K):
            pltpu.sync_copy(zrows, acc_sh.at[pl.ds(s * rps + k * _K, _K)])
        tail = rps % _K
        if tail:
            pltpu.sync_copy(zrows.at[pl.ds(0, tail)],
                            acc_sh.at[pl.ds(s * rps + (rps // _K) * _K,
                                            tail)])
        pltpu.sync_copy(bnc_v, den_sh.at[pl.ds(s * rps, rps)])
        if rem:
            @pl.when(s == 0)
            def _():
                pltpu.sync_copy(zrows.at[pl.ds(0, rem)],
                                acc_sh.at[pl.ds(_NS * rps, rem)])
                pltpu.sync_copy(bnc_v.at[pl.ds(0, rem)],
                                den_sh.at[pl.ds(_NS * rps, rem)])
        plsc.subcore_barrier()

        lane = lax.iota(jnp.int32, 16)

        def start_idx(i, t):
            pltpu.async_copy(src_hbm.at[pl.ds(base + i, 1)], si3[t], isem[t])
            pltpu.async_copy(dst_hbm.at[pl.ds(base + i, 1)], di3[t], isem[t])

        def wait_idx(t):
            pltpu.make_async_copy(src_hbm.at[pl.ds(0, 1)], si3[t],
                                  isem[t]).wait()
            pltpu.make_async_copy(dst_hbm.at[pl.ds(0, 1)], di3[t],
                                  isem[t]).wait()

        def start_scores(b, t):
            pltpu.async_copy(as_hbm.at[si3[t].at[0]], ag2[b], asem[b])
            pltpu.async_copy(ad_hbm.at[di3[t].at[0]], dg2[b], dsem[b])

        def wait_scores(b):
            pltpu.make_async_copy(as_hbm.at[si3[0].at[0]], ag2[b],
                                  asem[b]).wait()
            pltpu.make_async_copy(ad_hbm.at[di3[0].at[0]], dg2[b],
                                  dsem[b]).wait()

        def start_rows(b, t):
            pltpu.async_copy(h_hbm.at[si3[t].at[0]], rows2[b], gsem[b])

        def wait_rows(b):
            pltpu.make_async_copy(h_hbm.at[si3[0].at[0]], rows2[b],
                                  gsem[b]).wait()

        def start_scatter(b, t):
            pltpu.async_copy(rows2[b], acc_sh.at[di3[t].at[0]], ssem[b],
                             add=True)

        def wait_scatter(b):
            pltpu.make_async_copy(rows2[b], acc_sh.at[di3[0].at[0]],
                                  ssem[b]).wait()

        def start_den(b, t):
            pltpu.async_copy(w2[b], den_sh.at[di3[t].at[0]], wsem[b],
                             add=True)

        def wait_den(b):
            pltpu.make_async_copy(w2[b], den_sh.at[di3[0].at[0]],
                                  wsem[b]).wait()

        def weights(i, b, t):
            @plsc.parallel_loop(0, ngrp, unroll=2)
            def grp(j):
                e = ag2[b][pl.ds(j * 16, 16)] + dg2[b][pl.ds(j * 16, 16)]
                e = jnp.maximum(e, 0.2 * e)
                w = jnp.exp(e)
                eid = (base + i) * _K + j * 16 + lane
                w = jnp.where(eid < e_real, w, 0.0)
                w2[b][pl.ds(j * 16, 16)] = w

        def scale(b):
            rows_v, w_v = rows2[b], w2[b]

            @plsc.parallel_loop(0, _K, unroll=8)
            def body(eix):
                wvec = plsc.load_gather(
                    w_v, [jnp.full((16,), eix, jnp.int32)])
                for dd in range(d // 16):
                    sl = pl.ds(dd * 16, 16)
                    rows_v[eix, sl] = rows_v[eix, sl] * wvec

        start_idx(0, 0)
        wait_idx(0)
        start_scores(0, 0)
        start_rows(0, 0)
        start_idx(1, 1)

        def body_pair(i, carry):
            for u in range(6):
                ci = i + u
                b = u & 1
                bn = (u + 1) & 1
                t = u % 3
                tn = (u + 1) % 3
                tnn = (u + 2) % 3
                wait_scores(b)
                weights(ci, b, t)
                @pl.when(ci + 1 < cpw)
                def _():
                    wait_idx(tn)
                    start_scores(bn, tn)

                @pl.when(ci >= 1)
                def _():
                    wait_scatter(bn)
                    wait_den(bn)

                @pl.when(ci + 1 < cpw)
                def _():
                    start_rows(bn, tn)

                @pl.when(ci + 2 < cpw)
                def _():
                    start_idx(ci + 2, tnn)
                wait_rows(b)
                scale(b)
                start_scatter(b, t)
                start_den(b, t)
            return carry

        lax.fori_loop(0, cpw // 6, lambda k, cr: body_pair(k * 6, cr), 0)
        wait_scatter((cpw - 1) & 1)
        wait_den((cpw - 1) & 1)
        plsc.subcore_barrier()
        pltpu.sync_copy(acc_sh.at[pl.ds(s * rps, rps)],
                        out_hbm.at[c, pl.ds(s * rps, rps)])
        pltpu.sync_copy(den_sh.at[pl.ds(s * rps, rps)], bnc_v)
        pltpu.sync_copy(bnc_v, den_hbm.at[pl.ds(c * n + s * rps, rps)])
        if rem:
            @pl.when(s == 0)
            def _():
                pltpu.sync_copy(acc_sh.at[pl.ds(_NS * rps, rem)],
                                out_hbm.at[c, pl.ds(_NS * rps, rem)])
                pltpu.sync_copy(den_sh.at[pl.ds(_NS * rps, rem)],
                                bnc_v.at[pl.ds(0, rem)])
                pltpu.sync_copy(bnc_v.at[pl.ds(0, rem)],
                                den_hbm.at[pl.ds(c * n + _NS * rps, rem)])

    return ek


def kernel(x, edge_index, W1, att_src1, att_dst1, b1, W2, att_src2, att_dst2, b2):
    n, d_in = x.shape
    d_hid = W1.shape[1]
    d_out = W2.shape[1]
    e_edges = edge_index.shape[1]
    e_real = e_edges + n

    cpw = -(-e_real // (_NW * _K))
    cpw = max(6, -(-cpw // 6) * 6)
    total = _NW * cpw * _K
    pad = total - e_real

    loop = jnp.arange(n, dtype=jnp.int32)
    fill = jnp.arange(pad, dtype=jnp.int32) % n
    src = jnp.concatenate([edge_index[0].astype(jnp.int32), loop, fill])
    dst = jnp.concatenate([edge_index[1].astype(jnp.int32), loop, fill])
    src = src.reshape(total // _K, _K)
    dst = dst.reshape(total // _K, _K)

    blk = 2000
    ek = _make_edge_kernel(n, d_hid, cpw, e_real)

    h1, as1, ad1 = _mm_att(x, W1, att_src1.reshape(d_hid, 1),
                           att_dst1.reshape(d_hid, 1), blk)
    p1, dn1 = ek(h1, as1.reshape(n), ad1.reshape(n), src, dst)

    h2, as2, ad2 = _norm_relu_mm_att(p1, dn1.reshape(_NC, n, 1),
                                     b1.reshape(1, d_hid), W2,
                                     att_src2.reshape(d_out, 1),
                                     att_dst2.reshape(d_out, 1), blk)
    p2, dn2 = ek(h2, as2.reshape(n), ad2.reshape(n), src, dst)

    return _norm_bias(p2, dn2.reshape(_NC, n, 1), b2.reshape(1, d_out), blk)


# speedup vs baseline: 51.5193x; 1.0300x over previous
import functools

import jax
import jax.numpy as jnp
from jax import lax
from jax.experimental import pallas as pl
from jax.experimental.pallas import tpu as pltpu
from jax.experimental.pallas import tpu_sc as plsc

_NC = 2
_NS = 16
_NW = _NC * _NS
_K = 96


def _mm_att(x, W, a_src, a_dst, blk):
    n, d_in = x.shape
    d = W.shape[1]

    def body(x_ref, w_ref, s_ref, t_ref, h_ref, as_ref, ad_ref):
        h = jnp.dot(x_ref[...], w_ref[...], preferred_element_type=jnp.float32)
        h_ref[...] = h
        as_ref[...] = jnp.dot(h, s_ref[...], preferred_element_type=jnp.float32)
        ad_ref[...] = jnp.dot(h, t_ref[...], preferred_element_type=jnp.float32)

    return pl.pallas_call(
        body,
        grid=(n // blk,),
        in_specs=[
            pl.BlockSpec((blk, d_in), lambda i: (i, 0)),
            pl.BlockSpec((d_in, d), lambda i: (0, 0)),
            pl.BlockSpec((d, 1), lambda i: (0, 0)),
            pl.BlockSpec((d, 1), lambda i: (0, 0)),
        ],
        out_specs=[
            pl.BlockSpec((blk, d), lambda i: (i, 0)),
            pl.BlockSpec((blk, 1), lambda i: (i, 0)),
            pl.BlockSpec((blk, 1), lambda i: (i, 0)),
        ],
        out_shape=[
            jax.ShapeDtypeStruct((n, d), jnp.float32),
            jax.ShapeDtypeStruct((n, 1), jnp.float32),
            jax.ShapeDtypeStruct((n, 1), jnp.float32),
        ],
    )(x, W, a_src, a_dst)


def _norm_relu_mm_att(p, dn, b, W, a_src, a_dst, blk):
    _, n, d = p.shape
    d2 = W.shape[1]

    def body(p_ref, dn_ref, b_ref, w_ref, s_ref, t_ref, h_ref, as_ref, ad_ref):
        acc = p_ref[0] + p_ref[1]
        den = dn_ref[0] + dn_ref[1] + 1e-16
        z = jnp.maximum(acc / den + b_ref[...], 0.0)
        h = jnp.dot(z, w_ref[...], preferred_element_type=jnp.float32)
        h_ref[...] = h
        as_ref[...] = jnp.dot(h, s_ref[...], preferred_element_type=jnp.float32)
        ad_ref[...] = jnp.dot(h, t_ref[...], preferred_element_type=jnp.float32)

    return pl.pallas_call(
        body,
        grid=(n // blk,),
        in_specs=[
            pl.BlockSpec((2, blk, d), lambda i: (0, i, 0)),
            pl.BlockSpec((2, blk, 1), lambda i: (0, i, 0)),
            pl.BlockSpec((1, d), lambda i: (0, 0)),
            pl.BlockSpec((d, d2), lambda i: (0, 0)),
            pl.BlockSpec((d2, 1), lambda i: (0, 0)),
            pl.BlockSpec((d2, 1), lambda i: (0, 0)),
        ],
        out_specs=[
            pl.BlockSpec((blk, d2), lambda i: (i, 0)),
            pl.BlockSpec((blk, 1), lambda i: (i, 0)),
            pl.BlockSpec((blk, 1), lambda i: (i, 0)),
        ],
        out_shape=[
            jax.ShapeDtypeStruct((n, d2), jnp.float32),
            jax.ShapeDtypeStruct((n, 1), jnp.float32),
            jax.ShapeDtypeStruct((n, 1), jnp.float32),
        ],
    )(p, dn, b, W, a_src, a_dst)


def _norm_bias(p, dn, b, blk):
    _, n, d = p.shape

    def body(p_ref, dn_ref, b_ref, o_ref):
        den = dn_ref[0] + dn_ref[1] + 1e-16
        o_ref[...] = (p_ref[0] + p_ref[1]) / den + b_ref[...]

    return pl.pallas_call(
        body,
        grid=(n // blk,),
        in_specs=[
            pl.BlockSpec((2, blk, d), lambda i: (0, i, 0)),
            pl.BlockSpec((2, blk, 1), lambda i: (0, i, 0)),
            pl.BlockSpec((1, d), lambda i: (0, 0)),
        ],
        out_specs=pl.BlockSpec((blk, d), lambda i: (i, 0)),
        out_shape=jax.ShapeDtypeStruct((n, d), jnp.float32),
    )(p, dn, b)


@functools.lru_cache(maxsize=None)
def _make_edge_kernel(n, d, cpw, e_real):
    mesh = plsc.VectorSubcoreMesh(core_axis_name="c", subcore_axis_name="s")
    rps = (n // _NS) // 16 * 16
    rem = n - _NS * rps
    ngrp = _K // 16

    @functools.partial(
        pl.kernel,
        out_type=(
            jax.ShapeDtypeStruct((_NC, n, d), jnp.float32),
            jax.ShapeDtypeStruct((_NC * n,), jnp.float32),
        ),
        mesh=mesh,
        compiler_params=pltpu.CompilerParams(needs_layout_passes=False),
        scratch_types=[
            [pltpu.VMEM((1, _K), jnp.int32) for _ in range(6)],
            [pltpu.VMEM((1, _K), jnp.int32) for _ in range(6)],
            [pltpu.VMEM((_K,), jnp.float32) for _ in range(3)],
            [pltpu.VMEM((_K,), jnp.float32) for _ in range(3)],
            [pltpu.VMEM((_K,), jnp.float32) for _ in range(3)],
            [pltpu.VMEM((_K, d), jnp.float32) for _ in range(3)],
            pltpu.VMEM_SHARED((n, d), jnp.float32),
            pltpu.VMEM_SHARED((n,), jnp.float32),
            pltpu.VMEM((rps,), jnp.float32),
            [pltpu.SemaphoreType.DMA for _ in range(6)],
            [pltpu.SemaphoreType.DMA for _ in range(3)],
            [pltpu.SemaphoreType.DMA for _ in range(3)],
            [pltpu.SemaphoreType.DMA for _ in range(3)],
            [pltpu.SemaphoreType.DMA for _ in range(3)],
            [pltpu.SemaphoreType.DMA for _ in range(3)],
        ],
    )
    def ek(h_hbm, as_hbm, ad_hbm, src_hbm, dst_hbm,
           out_hbm, den_hbm,
           si3, di3, ag2, dg2, w2, rows2, acc_sh, den_sh, bnc_v,
           isem, asem, dsem, gsem, ssem, wsem):
        c = lax.axis_index("c")
        s = lax.axis_index("s")
        wid = c * _NS + s
        base = wid * cpw
        zrows = rows2[0]

        @plsc.parallel_loop(0, _K, unroll=4)
        def _(e):
            for dd in range(d // 16):
                zrows[e, pl.ds(dd * 16, 16)] = jnp.zeros((16,), jnp.float32)

        def zb(j, carry):
            bnc_v[pl.ds(j * 16, 16)] = jnp.zeros((16,), jnp.float32)
            return carry

        lax.fori_loop(0, rps // 16, zb, 0)
        for k in range(rps // _K):
            pltpu.sync_copy(zrows, acc_sh.at[pl.ds(s * rps + k * _K, _K)])
        tail = rps % _K
        if tail:
            pltpu.sync_copy(zrows.at[pl.ds(0, tail)],
                            acc_sh.at[pl.ds(s * rps + (rps // _K) * _K,
                                            tail)])
        pltpu.sync_copy(bnc_v, den_sh.at[pl.ds(s * rps, rps)])
        if rem:
            @pl.when(s == 0)
            def _():
                pltpu.sync_copy(zrows.at[pl.ds(0, rem)],
                                acc_sh.at[pl.ds(_NS * rps, rem)])
                pltpu.sync_copy(bnc_v.at[pl.ds(0, rem)],
                                den_sh.at[pl.ds(_NS * rps, rem)])
        plsc.subcore_barrier()

        lane = lax.iota(jnp.int32, 16)

        def start_idx(i, t):
            pltpu.async_copy(src_hbm.at[pl.ds(base + i, 1)], si3[t], isem[t])
            pltpu.async_copy(dst_hbm.at[pl.ds(base + i, 1)], di3[t], isem[t])

        def wait_idx(t):
            pltpu.make_async_copy(src_hbm.at[pl.ds(0, 1)], si3[t],
                                  isem[t]).wait()
            pltpu.make_async_copy(dst_hbm.at[pl.ds(0, 1)], di3[t],
                                  isem[t]).wait()

        def start_scores(b, t):
            pltpu.async_copy(as_hbm.at[si3[t].at[0]], ag2[b], asem[b])
            pltpu.async_copy(ad_hbm.at[di3[t].at[0]], dg2[b], dsem[b])

        def wait_scores(b):
            pltpu.make_async_copy(as_hbm.at[si3[0].at[0]], ag2[b],
                                  asem[b]).wait()
            pltpu.make_async_copy(ad_hbm.at[di3[0].at[0]], dg2[b],
                                  dsem[b]).wait()

        def start_rows(b, t):
            pltpu.async_copy(h_hbm.at[si3[t].at[0]], rows2[b], gsem[b])

        def wait_rows(b):
            pltpu.make_async_copy(h_hbm.at[si3[0].at[0]], rows2[b],
                                  gsem[b]).wait()

        def start_scatter(b, t):
            pltpu.async_copy(rows2[b], acc_sh.at[di3[t].at[0]], ssem[b],
                             add=True)

        def wait_scatter(b):
            pltpu.make_async_copy(rows2[b], acc_sh.at[di3[0].at[0]],
                                  ssem[b]).wait()

        def start_den(b, t):
            pltpu.async_copy(w2[b], den_sh.at[di3[t].at[0]], wsem[b],
                             add=True)

        def wait_den(b):
            pltpu.make_async_copy(w2[b], den_sh.at[di3[0].at[0]],
                                  wsem[b]).wait()

        def weights(i, b, t):
            @plsc.parallel_loop(0, ngrp, unroll=2)
            def grp(j):
                e = ag2[b][pl.ds(j * 16, 16)] + dg2[b][pl.ds(j * 16, 16)]
                e = jnp.maximum(e, 0.2 * e)
                w = jnp.exp(e)
                eid = (base + i) * _K + j * 16 + lane
                w = jnp.where(eid < e_real, w, 0.0)
                w2[b][pl.ds(j * 16, 16)] = w

        def scale(b):
            rows_v, w_v = rows2[b], w2[b]

            @plsc.parallel_loop(0, _K, unroll=8)
            def body(eix):
                wvec = plsc.load_gather(
                    w_v, [jnp.full((16,), eix, jnp.int32)])
                for dd in range(d // 16):
                    sl = pl.ds(dd * 16, 16)
                    rows_v[eix, sl] = rows_v[eix, sl] * wvec

        start_idx(0, 0)
        start_idx(1, 1)
        start_idx(2, 2)
        wait_idx(0)
        start_scores(0, 0)
        start_rows(0, 0)

        def body6(i, carry):
            for u in range(6):
                ci = i + u
                r = u % 3
                rn = (u + 1) % 3
                q = u
                qn = (u + 1) % 6
                qm = (u + 3) % 6
                wait_scores(r)
                weights(ci, r, q)

                @pl.when(ci >= 2)
                def _():
                    wait_scatter(rn)
                    wait_den(rn)

                @pl.when(ci + 1 < cpw)
                def _():
                    wait_idx(qn)
                    start_scores(rn, qn)
                    start_rows(rn, qn)

                @pl.when(ci + 3 < cpw)
                def _():
                    start_idx(ci + 3, qm)

                wait_rows(r)
                scale(r)
                start_scatter(r, q)
                start_den(r, q)
            return carry

        lax.fori_loop(0, cpw // 6, lambda k, cr: body6(k * 6, cr), 0)
        wait_scatter((cpw - 2) % 3)
        wait_den((cpw - 2) % 3)
        wait_scatter((cpw - 1) % 3)
        wait_den((cpw - 1) % 3)
        plsc.subcore_barrier()
        pltpu.sync_copy(acc_sh.at[pl.ds(s * rps, rps)],
                        out_hbm.at[c, pl.ds(s * rps, rps)])
        pltpu.sync_copy(den_sh.at[pl.ds(s * rps, rps)], bnc_v)
        pltpu.sync_copy(bnc_v, den_hbm.at[pl.ds(c * n + s * rps, rps)])
        if rem:
            @pl.when(s == 0)
            def _():
                pltpu.sync_copy(acc_sh.at[pl.ds(_NS * rps, rem)],
                                out_hbm.at[c, pl.ds(_NS * rps, rem)])
                pltpu.sync_copy(den_sh.at[pl.ds(_NS * rps, rem)],
                                bnc_v.at[pl.ds(0, rem)])
                pltpu.sync_copy(bnc_v.at[pl.ds(0, rem)],
                                den_hbm.at[pl.ds(c * n + _NS * rps, rem)])

    return ek


def kernel(x, edge_index, W1, att_src1, att_dst1, b1, W2, att_src2, att_dst2, b2):
    n, d_in = x.shape
    d_hid = W1.shape[1]
    d_out = W2.shape[1]
    e_edges = edge_index.shape[1]
    e_real = e_edges + n

    cpw = -(-e_real // (_NW * _K))
    cpw = max(6, -(-cpw // 6) * 6)
    total = _NW * cpw * _K
    pad = total - e_real

    loop = jnp.arange(n, dtype=jnp.int32)
    fill = jnp.arange(pad, dtype=jnp.int32) % n
    src = jnp.concatenate([edge_index[0].astype(jnp.int32), loop, fill])
    dst = jnp.concatenate([edge_index[1].astype(jnp.int32), loop, fill])
    src = src.reshape(total // _K, _K)
    dst = dst.reshape(total // _K, _K)

    blk = 2000
    ek = _make_edge_kernel(n, d_hid, cpw, e_real)

    h1, as1, ad1 = _mm_att(x, W1, att_src1.reshape(d_hid, 1),
                           att_dst1.reshape(d_hid, 1), blk)
    p1, dn1 = ek(h1, as1.reshape(n), ad1.reshape(n), src, dst)

    h2, as2, ad2 = _norm_relu_mm_att(p1, dn1.reshape(_NC, n, 1),
                                     b1.reshape(1, d_hid), W2,
                                     att_src2.reshape(d_out, 1),
                                     att_dst2.reshape(d_out, 1), blk)
    p2, dn2 = ek(h2, as2.reshape(n), ad2.reshape(n), src, dst)

    return _norm_bias(p2, dn2.reshape(_NC, n, 1), b2.reshape(1, d_out), blk)
